# baseline (speedup 1.0000x reference)
import functools

import jax
import jax.numpy as jnp
from jax import lax
from jax.experimental import pallas as pl
from jax.experimental.pallas import tpu as pltpu
from jax.experimental.pallas import tpu_sc as plsc

NC = 2
NS = 16
NW = NC * NS
L = 16

C = 20
H = 16
ROW = C * H

KR = 640
TB = 1024


def _sc_gather_body(idx_hbm, w_hbm, b_hbm, wbt_hbm, idx_v, wf_v, bf_v, wb_v):
    rows = idx_hbm.shape[0]
    rows_per_w = rows // NW
    chunks = rows_per_w // KR
    wid = lax.axis_index("s") * NC + lax.axis_index("c")
    base0 = wid * rows_per_w

    pltpu.sync_copy(w_hbm, wf_v)
    pltpu.sync_copy(b_hbm, bf_v)

    def chunk_body(ci, carry):
        base = base0 + ci * KR
        pltpu.sync_copy(idx_hbm.at[pl.ds(base, KR)], idx_v)

        def group_body(g, gcarry):
            addr = idx_v[pl.ds(g * L, L)] * H
            for h in range(H):
                wb_v[h, pl.ds(g * L, L)] = plsc.load_gather(wf_v, [addr + h])
                wb_v[H + h, pl.ds(g * L, L)] = plsc.load_gather(
                    bf_v, [addr + h])
            return gcarry

        lax.fori_loop(0, KR // L, group_body, 0, unroll=False)
        pltpu.sync_copy(wb_v, wbt_hbm.at[:, pl.ds(base, KR)])
        return carry

    lax.fori_loop(0, chunks, chunk_body, 0, unroll=False)


def _sc_gather(idx, w, b):
    rows = idx.shape[0]
    mesh = plsc.VectorSubcoreMesh(core_axis_name="c", subcore_axis_name="s")
    run = pl.kernel(
        _sc_gather_body,
        out_type=jax.ShapeDtypeStruct((2 * H, rows), jnp.float32),
        mesh=mesh,
        compiler_params=pltpu.CompilerParams(needs_layout_passes=False),
        scratch_types=[
            pltpu.VMEM((KR,), jnp.int32),
            pltpu.VMEM((168 * H,), jnp.float32),
            pltpu.VMEM((168 * H,), jnp.float32),
            pltpu.VMEM((2 * H, KR), jnp.float32),
        ],
    )
    return run(idx, w.reshape(-1), b.reshape(-1))


def _tc_affine_body(x_ref, wb_ref, s_ref, o_ref):
    w16 = wb_ref[0:H, :]
    b16 = wb_ref[H:2 * H, :]
    sel = s_ref[...]
    dn = (((0,), (0,)), ((), ()))
    wt = lax.dot_general(w16, sel, dn, precision=lax.Precision.HIGHEST,
                         preferred_element_type=jnp.float32)
    bt = lax.dot_general(b16, sel, dn, precision=lax.Precision.HIGHEST,
                         preferred_element_type=jnp.float32)
    o_ref[...] = x_ref[...] * wt + bt


def _tc_affine(x2, wbt, sel):
    rows = x2.shape[0]
    grid = (rows // TB,)
    return pl.pallas_call(
        _tc_affine_body,
        grid=grid,
        in_specs=[
            pl.BlockSpec((TB, ROW), lambda i: (i, 0)),
            pl.BlockSpec((2 * H, TB), lambda i: (0, i)),
            pl.BlockSpec((H, ROW), lambda i: (0, 0)),
        ],
        out_specs=pl.BlockSpec((TB, ROW), lambda i: (i, 0)),
        out_shape=jax.ShapeDtypeStruct((rows, ROW), jnp.float32),
    )(x2, wbt, sel)


@functools.partial(jax.jit, static_argnums=())
def kernel(x, periodic_indices, w, b):
    n, t, c, h = x.shape
    rows = n * t
    x2 = x.reshape(rows, c * h)
    idx = periodic_indices.reshape(rows).astype(jnp.int32)
    sel = jnp.tile(jnp.eye(H, dtype=jnp.float32), (1, C))

    wbt = _sc_gather(idx, w, b)
    out = _tc_affine(x2, wbt, sel)
    return out.reshape(n, t, c, h)


# speedup vs baseline: 7.4417x; 7.4417x over previous
import functools

import jax
import jax.numpy as jnp
from jax import lax
from jax.experimental import pallas as pl
from jax.experimental.pallas import tpu as pltpu
from jax.experimental.pallas import tpu_sc as plsc

NC = 2
NS = 16
NW = NC * NS
L = 16

C = 20
H = 16


def _sc_gather_body(idx_hbm, w_hbm, b_hbm, wb_hbm, idx_v, wf_v, bf_v, wb_v):
    T, N = idx_hbm.shape
    nv = N // L
    wid = lax.axis_index("s") * NC + lax.axis_index("c")

    pltpu.sync_copy(w_hbm, wf_v)
    pltpu.sync_copy(b_hbm, bf_v)

    def t_body(k, carry):
        t = wid + k * NW

        @pl.when(t < T)
        def _():
            pltpu.sync_copy(idx_hbm.at[t], idx_v)

            def v_body(v, vcarry):
                a16 = idx_v[pl.ds(v * L, L)] * H
                for h in range(H):
                    wb_v[h, pl.ds(v * L, L)] = plsc.load_gather(
                        wf_v, [a16 + h])
                    wb_v[H + h, pl.ds(v * L, L)] = plsc.load_gather(
                        bf_v, [a16 + h])
                return vcarry

            lax.fori_loop(0, nv, v_body, 0, unroll=False)
            pltpu.sync_copy(wb_v, wb_hbm.at[t])
        return carry

    lax.fori_loop(0, (T + NW - 1) // NW, t_body, 0, unroll=False)


def _sc_gather(idxt, w, b):
    T, N = idxt.shape
    mesh = plsc.VectorSubcoreMesh(core_axis_name="c", subcore_axis_name="s")
    run = pl.kernel(
        _sc_gather_body,
        out_type=jax.ShapeDtypeStruct((T, 2 * H, N), jnp.float32),
        mesh=mesh,
        compiler_params=pltpu.CompilerParams(needs_layout_passes=False),
        scratch_types=[
            pltpu.VMEM((N,), jnp.int32),
            pltpu.VMEM((168 * H,), jnp.float32),
            pltpu.VMEM((168 * H,), jnp.float32),
            pltpu.VMEM((2 * H, N), jnp.float32),
        ],
    )
    return run(idxt, w.reshape(-1), b.reshape(-1))


def _tc_affine_body(x_ref, wb_ref, o_ref):
    w16 = wb_ref[0, 0:H, :]
    b16 = wb_ref[0, H:2 * H, :]
    for c in range(C):
        o_ref[0, c] = x_ref[0, c] * w16 + b16


def _tc_affine(xt, wbsel):
    T, c, h, N = xt.shape
    return pl.pallas_call(
        _tc_affine_body,
        grid=(T,),
        in_specs=[
            pl.BlockSpec((1, C, H, N), lambda i: (i, 0, 0, 0)),
            pl.BlockSpec((1, 2 * H, N), lambda i: (i, 0, 0)),
        ],
        out_specs=pl.BlockSpec((1, C, H, N), lambda i: (i, 0, 0, 0)),
        out_shape=jax.ShapeDtypeStruct((T, C, H, N), jnp.float32),
    )(xt, wbsel)


@functools.partial(jax.jit, static_argnums=())
def kernel(x, periodic_indices, w, b):
    n, t, c, h = x.shape
    xt = jnp.transpose(x, (1, 2, 3, 0))
    idxt = periodic_indices.T.astype(jnp.int32)

    wbsel = _sc_gather(idxt, w, b)
    zt = _tc_affine(xt, wbsel)
    return jnp.transpose(zt, (3, 0, 1, 2))


# speedup vs baseline: 10.6604x; 1.4325x over previous
import functools

import jax
import jax.numpy as jnp
from jax import lax
from jax.experimental import pallas as pl
from jax.experimental.pallas import tpu as pltpu
from jax.experimental.pallas import tpu_sc as plsc

NC = 2
NS = 16
NW = NC * NS
L = 16

C = 20
H = 16

NCHUNK = 4
TCB = 2


def _make_sc_body(t_lo, t_hi):
    def body(idx_hbm, w_hbm, b_hbm, wb_hbm, idx_v, wf_v, bf_v, wb_v):
        T, N = idx_hbm.shape
        nv = N // L
        wid = lax.axis_index("s") * NC + lax.axis_index("c")

        pltpu.sync_copy(w_hbm, wf_v)
        pltpu.sync_copy(b_hbm, bf_v)

        def t_body(k, carry):
            t = t_lo + wid + k * NW

            @pl.when(t < t_hi)
            def _():
                pltpu.sync_copy(idx_hbm.at[t], idx_v)

                def v_body(v, vcarry):
                    a16 = idx_v[pl.ds(v * L, L)] * H
                    for h in range(H):
                        wb_v[h, pl.ds(v * L, L)] = plsc.load_gather(
                            wf_v, [a16 + h])
                        wb_v[H + h, pl.ds(v * L, L)] = plsc.load_gather(
                            bf_v, [a16 + h])
                    return vcarry

                lax.fori_loop(0, nv, v_body, 0, unroll=False)
                pltpu.sync_copy(wb_v, wb_hbm.at[t - t_lo])
            return carry

        lax.fori_loop(0, (t_hi - t_lo + NW - 1) // NW, t_body, 0,
                      unroll=False)

    return body


def _sc_gather_chunk(idxt, w, b, t_lo, t_hi):
    T, N = idxt.shape
    mesh = plsc.VectorSubcoreMesh(core_axis_name="c", subcore_axis_name="s")
    run = pl.kernel(
        _make_sc_body(t_lo, t_hi),
        out_type=jax.ShapeDtypeStruct((t_hi - t_lo, 2 * H, N), jnp.float32),
        mesh=mesh,
        compiler_params=pltpu.CompilerParams(needs_layout_passes=False),
        scratch_types=[
            pltpu.VMEM((N,), jnp.int32),
            pltpu.VMEM((168 * H,), jnp.float32),
            pltpu.VMEM((168 * H,), jnp.float32),
            pltpu.VMEM((2 * H, N), jnp.float32),
        ],
    )
    return run(idxt, w.reshape(-1), b.reshape(-1))


def _tc_affine_body(x_ref, wb_ref, z_ref, o_ref):
    for s in range(TCB):
        w16 = wb_ref[s, 0:H, :]
        b16 = wb_ref[s, H:2 * H, :]
        for c in range(C):
            o_ref[s, c] = x_ref[s, c] * w16 + b16


def _tc_affine_chunk(xt, wb, z_prev, t_lo, t_hi):
    T, c, h, N = xt.shape
    grid = ((t_hi - t_lo) // TCB,)
    blk = t_lo // TCB
    kwargs = {}
    if z_prev is not None:
        kwargs = dict(input_output_aliases={2: 0})
    else:
        z_prev = jnp.zeros((1, 1), dtype=jnp.float32)
    return pl.pallas_call(
        _tc_affine_body,
        grid=grid,
        in_specs=[
            pl.BlockSpec((TCB, C, H, N), lambda i: (blk + i, 0, 0, 0)),
            pl.BlockSpec((TCB, 2 * H, N), lambda i: (i, 0, 0)),
            pl.BlockSpec(memory_space=pl.ANY),
        ],
        out_specs=pl.BlockSpec((TCB, C, H, N), lambda i: (blk + i, 0, 0, 0)),
        out_shape=jax.ShapeDtypeStruct((T, C, H, N), jnp.float32),
        **kwargs,
    )(xt, wb, z_prev)


@functools.partial(jax.jit, static_argnums=())
def kernel(x, periodic_indices, w, b):
    n, t, c, h = x.shape
    xt = jnp.transpose(x, (1, 2, 3, 0))
    idxt = periodic_indices.T.astype(jnp.int32)

    bounds = [t * i // NCHUNK for i in range(NCHUNK + 1)]
    wbs = [_sc_gather_chunk(idxt, w, b, lo, hi)
           for lo, hi in zip(bounds[:-1], bounds[1:])]

    z = None
    for (lo, hi), wb in zip(zip(bounds[:-1], bounds[1:]), wbs):
        z = _tc_affine_chunk(xt, wb, z, lo, hi)

    return jnp.transpose(z, (3, 0, 1, 2))
